# baseline (speedup 1.0000x reference)
import jax
import jax.numpy as jnp
from jax import lax
from jax.experimental import pallas as pl
from jax.experimental.pallas import tpu as pltpu
from jax.experimental.pallas import tpu_sc as plsc

LANES = 16
NUM_CORES = 2
NUM_SUBCORES = 16
NUM_WORKERS = NUM_CORES * NUM_SUBCORES


def _svd_body(users_hbm, movies_hbm, U_hbm, M_hbm, ub_hbm, mb_hbm, out_hbm,
              uidx_v, midx_v, urows_v, mrows_v, ubias_v, mbias_v, out_v, sem):
    b_per_w = uidx_v.shape[0]
    latent = urows_v.shape[1]
    wid = lax.axis_index("s") * NUM_CORES + lax.axis_index("c")
    base = wid * b_per_w

    pltpu.sync_copy(users_hbm.at[pl.ds(base, b_per_w)], uidx_v)
    pltpu.sync_copy(movies_hbm.at[pl.ds(base, b_per_w)], midx_v)

    copies = [
        pltpu.async_copy(U_hbm.at[uidx_v], urows_v, sem),
        pltpu.async_copy(M_hbm.at[midx_v], mrows_v, sem),
        pltpu.async_copy(ub_hbm.at[uidx_v], ubias_v, sem),
        pltpu.async_copy(mb_hbm.at[midx_v], mbias_v, sem),
    ]
    for c in copies:
        c.wait()

    lanes = lax.iota(jnp.int32, LANES)

    def group(g, carry):
        row0 = g * LANES
        rows = row0 + lanes
        acc = ubias_v[pl.ds(row0, LANES)] + mbias_v[pl.ds(row0, LANES)]
        for d in range(latent):
            dcol = jnp.full((LANES,), d, jnp.int32)
            uc = plsc.load_gather(urows_v, [rows, dcol])
            mc = plsc.load_gather(mrows_v, [rows, dcol])
            acc = acc + uc * mc
        out_v[pl.ds(row0, LANES)] = acc
        return carry

    lax.fori_loop(0, b_per_w // LANES, group, 0)

    pltpu.sync_copy(out_v, out_hbm.at[pl.ds(base, b_per_w)])


def kernel(users, movies, U, M, user_bias, movie_bias):
    B = users.shape[0]
    b_per_w = B // NUM_WORKERS
    latent = U.shape[1]
    users = users.astype(jnp.int32)
    movies = movies.astype(jnp.int32)
    mesh = plsc.VectorSubcoreMesh(core_axis_name="c", subcore_axis_name="s")
    k = pl.kernel(
        _svd_body,
        out_type=jax.ShapeDtypeStruct((B,), jnp.float32),
        mesh=mesh,
        compiler_params=pltpu.CompilerParams(
            needs_layout_passes=False, use_tc_tiling_on_sc=False),
        scratch_types=[
            pltpu.VMEM((b_per_w,), jnp.int32),
            pltpu.VMEM((b_per_w,), jnp.int32),
            pltpu.VMEM((b_per_w, latent), jnp.float32),
            pltpu.VMEM((b_per_w, latent), jnp.float32),
            pltpu.VMEM((b_per_w,), jnp.float32),
            pltpu.VMEM((b_per_w,), jnp.float32),
            pltpu.VMEM((b_per_w,), jnp.float32),
            pltpu.SemaphoreType.DMA,
        ],
    )
    return k(users, movies, U, M, user_bias, movie_bias)


# speedup vs baseline: 1.0138x; 1.0138x over previous
import jax
import jax.numpy as jnp
from jax import lax
from jax.experimental import pallas as pl
from jax.experimental.pallas import tpu as pltpu
from jax.experimental.pallas import tpu_sc as plsc

LANES = 16
NUM_CORES = 2
NUM_SUBCORES = 16
NUM_WORKERS = NUM_CORES * NUM_SUBCORES
PACK = 4
LATENT = 32
CHUNK = 256


def _svd_body(users_hbm, movies_hbm, U_hbm, M_hbm, ub_hbm, mb_hbm, out_hbm,
              uidx_v, midx_v, uq_v, mq_v, uch_v, mch_v,
              ubias_v, mbias_v, out_v, sem):
    b_per_w = uidx_v.shape[0]
    wid = lax.axis_index("s") * NUM_CORES + lax.axis_index("c")
    base = wid * b_per_w

    pltpu.sync_copy(users_hbm.at[pl.ds(base, b_per_w)], uidx_v)
    pltpu.sync_copy(movies_hbm.at[pl.ds(base, b_per_w)], midx_v)

    bias_copies = [
        pltpu.async_copy(ub_hbm.at[uidx_v], ubias_v, sem),
        pltpu.async_copy(mb_hbm.at[midx_v], mbias_v, sem),
    ]

    def fill_q(g, _):
        sl = pl.ds(g * LANES, LANES)
        uq_v[sl] = lax.shift_right_logical(uidx_v[sl], 2)
        mq_v[sl] = lax.shift_right_logical(midx_v[sl], 2)
        return 0

    lax.fori_loop(0, b_per_w // LANES, fill_q, 0, unroll=4)

    for cp in bias_copies:
        cp.wait()

    lanes = lax.iota(jnp.int32, LANES)

    def do_chunk(c, _):
        cb = c * CHUNK
        copies = [
            pltpu.async_copy(U_hbm.at[uq_v.at[pl.ds(cb, CHUNK)]], uch_v, sem),
            pltpu.async_copy(M_hbm.at[mq_v.at[pl.ds(cb, CHUNK)]], mch_v, sem),
        ]
        for cp in copies:
            cp.wait()

        def group(g, _):
            b0 = cb + g * LANES
            sl = pl.ds(b0, LANES)
            uvec = uidx_v[sl]
            mvec = midx_v[sl]
            uoff = (uvec & (PACK - 1)) * LATENT
            moff = (mvec & (PACK - 1)) * LATENT
            rows = g * LANES + lanes
            acc = ubias_v[sl] + mbias_v[sl]
            for d0 in range(LATENT):
                rot = (lanes + d0) & (LATENT - 1)
                uc = plsc.load_gather(uch_v, [rows, uoff + rot])
                mc = plsc.load_gather(mch_v, [rows, moff + rot])
                acc = acc + uc * mc
            out_v[sl] = acc
            return 0

        lax.fori_loop(0, CHUNK // LANES, group, 0)
        return 0

    lax.fori_loop(0, b_per_w // CHUNK, do_chunk, 0)

    pltpu.sync_copy(out_v, out_hbm.at[pl.ds(base, b_per_w)])


def kernel(users, movies, U, M, user_bias, movie_bias):
    B = users.shape[0]
    b_per_w = B // NUM_WORKERS
    latent = U.shape[1]
    users = users.astype(jnp.int32)
    movies = movies.astype(jnp.int32)
    U128 = U.reshape(U.shape[0] * latent // 128, 128)
    M128 = M.reshape(M.shape[0] * latent // 128, 128)
    mesh = plsc.VectorSubcoreMesh(core_axis_name="c", subcore_axis_name="s")
    k = pl.kernel(
        _svd_body,
        out_type=jax.ShapeDtypeStruct((B,), jnp.float32),
        mesh=mesh,
        compiler_params=pltpu.CompilerParams(
            needs_layout_passes=False, use_tc_tiling_on_sc=True),
        scratch_types=[
            pltpu.VMEM((b_per_w,), jnp.int32),
            pltpu.VMEM((b_per_w,), jnp.int32),
            pltpu.VMEM((b_per_w,), jnp.int32),
            pltpu.VMEM((b_per_w,), jnp.int32),
            pltpu.VMEM((CHUNK, 128), jnp.float32),
            pltpu.VMEM((CHUNK, 128), jnp.float32),
            pltpu.VMEM((b_per_w,), jnp.float32),
            pltpu.VMEM((b_per_w,), jnp.float32),
            pltpu.VMEM((b_per_w,), jnp.float32),
            pltpu.SemaphoreType.DMA,
        ],
    )
    return k(users, movies, U128, M128, user_bias, movie_bias)
